# Initial kernel scaffold; baseline (speedup 1.0000x reference)
import jax
import jax.numpy as jnp
from jax.experimental import pallas as pl


def kernel(input_state, candidate_docs, use_actor_policy_net, W1, b1, W2, b2, W3, b3):
    raise NotImplementedError("write your pallas kernel here")


# speedup vs baseline: 1.2844x; 1.2844x over previous
import functools

import jax
import jax.numpy as jnp
from jax import lax
from jax.experimental import pallas as pl
from jax.experimental.pallas import tpu as pltpu
from jax.experimental.pallas import tpu_sc as plsc

K_NN = 128
SLATE = 5
DOC_DIM = 20
N_DOCS = 1_000_000
BLK = 64
N_BLOCKS = N_DOCS // BLK
CHUNK = 8000
N_CHUNK = N_DOCS // CHUNK
BPC = CHUNK // BLK
S_SEL = 192
POOL = S_SEL * BLK
ROW_W = BLK * DOC_DIM

_NC, _NS = 2, 16
_NW = _NC * _NS
_GATHER_ROWS = SLATE * S_SEL
_GATHER_PAD = 1024
_RPW = _GATHER_PAD // _NW


def _mlp_body(x_ref, w1_ref, b1_ref, w2_ref, b2_ref, w3_ref, b3_ref, out_ref):
    h = jnp.dot(x_ref[...], w1_ref[...], preferred_element_type=jnp.float32)
    h = jax.nn.leaky_relu(h + b1_ref[...])
    h = jnp.dot(h, w2_ref[...], preferred_element_type=jnp.float32)
    h = jax.nn.leaky_relu(h + b2_ref[...])
    h = jnp.dot(h, w3_ref[...], preferred_element_type=jnp.float32)
    h = jax.nn.leaky_relu(h + b3_ref[...])
    out_ref[...] = h


def _mlp(x, w1, b1, w2, b2, w3, b3):
    return pl.pallas_call(
        _mlp_body,
        out_shape=jax.ShapeDtypeStruct((1, SLATE * DOC_DIM), jnp.float32),
    )(x, w1, b1, w2, b2, w3, b3)


def _score_body(docs_ref, pmat_ref, ones_ref, out_ref):
    docs = docs_ref[...]
    dots = jnp.dot(docs, pmat_ref[...],
                   preferred_element_type=jnp.float32)
    cn = jnp.dot(docs * docs, ones_ref[...],
                 preferred_element_type=jnp.float32)
    s = cn + dots
    bmin = jnp.min(s.reshape(BPC, BLK, 128), axis=1)
    out_ref[...] = bmin[:, :8].T.reshape(1, 8, BPC)


def _score(docs, pmat, ones_m):
    return pl.pallas_call(
        _score_body,
        grid=(N_CHUNK,),
        in_specs=[
            pl.BlockSpec((CHUNK, DOC_DIM), lambda i: (i, 0)),
            pl.BlockSpec((DOC_DIM, 128), lambda i: (0, 0)),
            pl.BlockSpec((DOC_DIM, 128), lambda i: (0, 0)),
        ],
        out_specs=pl.BlockSpec((1, 8, BPC), lambda i: (i, 0, 0)),
        out_shape=jax.ShapeDtypeStruct((N_CHUNK, 8, BPC), jnp.float32),
    )(docs, pmat, ones_m)


def _select_body(bmin_ref, out_ref):
    d = bmin_ref[...]
    cpos = lax.broadcasted_iota(jnp.int32, (N_CHUNK, 8, BPC), 0)
    bpos = lax.broadcasted_iota(jnp.int32, (N_CHUNK, 8, BPC), 2)
    pos = cpos * BPC + bpos
    lane = lax.broadcasted_iota(jnp.int32, (1, 8, S_SEL), 2)

    def step(k, carry):
        m_prev, p_prev, acc = carry
        mask = (d > m_prev) | ((d == m_prev) & (pos > p_prev))
        dm = jnp.where(mask, d, jnp.inf)
        m = jnp.min(dm, axis=(0, 2), keepdims=True)
        sel = jnp.min(jnp.where(dm == m, pos, N_BLOCKS),
                      axis=(0, 2), keepdims=True).astype(jnp.int32)
        acc = acc + jnp.where(lane == k, sel, 0)
        return m, sel, acc

    m0 = jnp.full((1, 8, 1), -jnp.inf, jnp.float32)
    p0 = jnp.full((1, 8, 1), -1, jnp.int32)
    acc0 = jnp.zeros((1, 8, S_SEL), jnp.int32)
    _, _, acc = lax.fori_loop(0, S_SEL, step, (m0, p0, acc0))
    out_ref[...] = acc[0]


def _select(bmin):
    return pl.pallas_call(
        _select_body,
        out_shape=jax.ShapeDtypeStruct((8, S_SEL), jnp.int32),
    )(bmin)


def _gather_blocks_body(table_hbm, idx_hbm, out_hbm, idx_v, rows_v, sem):
    wid = lax.axis_index("s") * _NC + lax.axis_index("c")
    base = wid * _RPW
    pltpu.sync_copy(idx_hbm.at[pl.ds(base, _RPW)], idx_v)
    pltpu.async_copy(table_hbm.at[idx_v], rows_v, sem).wait()
    pltpu.sync_copy(rows_v, out_hbm.at[pl.ds(base, _RPW)])


def _gather_blocks(table, idx_full):
    gather = functools.partial(
        pl.kernel,
        out_type=jax.ShapeDtypeStruct((_GATHER_PAD, ROW_W), jnp.float32),
        mesh=plsc.VectorSubcoreMesh(core_axis_name="c", subcore_axis_name="s"),
        scratch_types=[
            pltpu.VMEM((_RPW,), jnp.int32),
            pltpu.VMEM((_RPW, ROW_W), jnp.float32),
            pltpu.SemaphoreType.DMA,
        ],
    )(_gather_blocks_body)
    return gather(table, idx_full)


_R = POOL // 128


def _final_body(pool3_ref, pool2_ref, proto_ref, gmat_ref, gflat_ref,
                outd_ref, outi_ref):
    diff = pool3_ref[...] - proto_ref[...]
    dmat = jnp.sqrt(jnp.sum(diff * diff, axis=2))
    gmat = gmat_ref[0]
    lane = lax.broadcasted_iota(jnp.int32, (1, K_NN), 1)
    big = jnp.int32(0x7FFFFFFF)

    def step(k, carry):
        m_prev, g_prev, acc = carry
        mask = (dmat > m_prev) | ((dmat == m_prev) & (gmat > g_prev))
        dm = jnp.where(mask, dmat, jnp.inf)
        m = jnp.min(dm)
        g = jnp.min(jnp.where(dm == m, gmat, big))
        acc = acc + jnp.where(lane == k, g, 0)
        return m, g, acc

    _, _, acc = lax.fori_loop(
        0, K_NN, step,
        (jnp.float32(-jnp.inf), jnp.int32(-1),
         jnp.zeros((1, K_NN), jnp.int32)))

    gsel = jnp.transpose(acc)
    pm = jnp.where(gsel == gflat_ref[0], 1.0, 0.0)
    outd_ref[...] = jnp.dot(pm, pool2_ref[...],
                            preferred_element_type=jnp.float32)
    outi_ref[...] = acc.reshape(1, 1, K_NN)


def _final(pool3, pool2, protos3, gmat3, gflat3):
    return pl.pallas_call(
        _final_body,
        grid=(SLATE,),
        in_specs=[
            pl.BlockSpec((_R, 128, DOC_DIM), lambda p: (p, 0, 0)),
            pl.BlockSpec((POOL, DOC_DIM), lambda p: (p, 0)),
            pl.BlockSpec((1, 1, DOC_DIM), lambda p: (p, 0, 0)),
            pl.BlockSpec((1, _R, 128), lambda p: (p, 0, 0)),
            pl.BlockSpec((1, 1, POOL), lambda p: (p, 0, 0)),
        ],
        out_specs=[
            pl.BlockSpec((K_NN, DOC_DIM), lambda p: (p, 0)),
            pl.BlockSpec((1, 1, K_NN), lambda p: (p, 0, 0)),
        ],
        out_shape=[
            jax.ShapeDtypeStruct((SLATE * K_NN, DOC_DIM), jnp.float32),
            jax.ShapeDtypeStruct((SLATE, 1, K_NN), jnp.int32),
        ],
    )(pool3, pool2, protos3, gmat3, gflat3)


def kernel(input_state, candidate_docs, use_actor_policy_net,
           W1, b1, W2, b2, W3, b3):
    del use_actor_policy_net
    x = input_state.reshape(1, DOC_DIM)
    p100 = _mlp(x, W1, b1.reshape(1, -1), W2, b2.reshape(1, -1),
                W3, b3.reshape(1, -1))
    protos = p100.reshape(SLATE, DOC_DIM)

    pmat = jnp.zeros((DOC_DIM, 128), jnp.float32)
    pmat = pmat.at[:, :SLATE].set(-2.0 * protos.T)
    ones_m = jnp.ones((DOC_DIM, 128), jnp.float32)
    bmin = _score(candidate_docs, pmat, ones_m)

    sel8 = _select(bmin)
    sel = sel8[:SLATE]

    idx_full = jnp.concatenate(
        [sel.reshape(-1), jnp.arange(_GATHER_PAD - _GATHER_ROWS,
                                     dtype=jnp.int32)])
    table = candidate_docs.reshape(N_BLOCKS, ROW_W)
    gathered = _gather_blocks(table, idx_full)
    pool2 = gathered.reshape(_GATHER_PAD * BLK, DOC_DIM)
    pool3 = gathered.reshape(_GATHER_PAD * BLK // 128, 128, DOC_DIM)

    off = jnp.arange(BLK, dtype=jnp.int32)
    gidx5 = sel[:, :, None] * BLK + off[None, None, :]
    gmat3 = gidx5.reshape(SLATE, _R, 128)
    gflat3 = gidx5.reshape(SLATE, 1, POOL)

    outd, outi = _final(pool3, pool2, protos.reshape(SLATE, 1, DOC_DIM),
                        gmat3, gflat3)
    return outd, outi.reshape(SLATE * K_NN)
